# baseline (speedup 1.0000x reference)
import functools

import jax
import jax.numpy as jnp
from jax.experimental import pallas as pl
from jax.experimental.pallas import tpu as pltpu

NUM_EXPERTS = 64
TOP_K = 2


def _top2(scores):
    i1 = jnp.argmax(scores, axis=-1)
    m1 = jnp.max(scores, axis=-1)
    cols = jax.lax.broadcasted_iota(jnp.int32, scores.shape, 1)
    masked = jnp.where(cols == i1[:, None], -jnp.inf, scores)
    i2 = jnp.argmax(masked, axis=-1)
    m2 = jnp.max(masked, axis=-1)
    return (jnp.stack([m1, m2], axis=-1),
            jnp.stack([i1, i2], axis=-1).astype(jnp.int32))


def _router_block(x0_ref, x1_ref, wt_ref, w_out_ref, i_out_ref, s_out_ref):
    wt = wt_ref[...]
    for h, x_ref in enumerate((x0_ref, x1_ref)):
        logits = jnp.dot(x_ref[...], wt, preferred_element_type=jnp.float32)
        m = jnp.max(logits, axis=-1, keepdims=True)
        e = jnp.exp(logits - m)
        s = jnp.sum(e, axis=-1, keepdims=True)
        scores = e / s
        s_out_ref[h] = scores
        w, i = _top2(scores)
        w_out_ref[h] = w
        i_out_ref[h] = i


@functools.partial(jax.jit, static_argnames=())
def kernel(x, W):
    n_tokens, d_model = x.shape
    wt = W.T
    bt = 1024
    half = n_tokens // 2
    nsteps = half // bt
    grid = (nsteps,)
    weights, indices, scores = pl.pallas_call(
        _router_block,
        grid=grid,
        in_specs=[
            pl.BlockSpec((bt, d_model), lambda i: (i, 0)),
            pl.BlockSpec((bt, d_model), lambda i: (i + nsteps, 0)),
            pl.BlockSpec((d_model, NUM_EXPERTS), lambda i: (0, 0)),
        ],
        out_specs=[
            pl.BlockSpec((2, bt, TOP_K), lambda i: (0, i, 0)),
            pl.BlockSpec((2, bt, TOP_K), lambda i: (0, i, 0)),
            pl.BlockSpec((2, bt, NUM_EXPERTS), lambda i: (0, i, 0)),
        ],
        out_shape=[
            jax.ShapeDtypeStruct((2, half, TOP_K), jnp.float32),
            jax.ShapeDtypeStruct((2, half, TOP_K), jnp.int32),
            jax.ShapeDtypeStruct((2, half, NUM_EXPERTS), jnp.float32),
        ],
        compiler_params=pltpu.CompilerParams(
            dimension_semantics=("arbitrary",),
        ),
    )(x, x, wt)
    return (weights.reshape(n_tokens, TOP_K),
            indices.reshape(n_tokens, TOP_K),
            scores.reshape(n_tokens, NUM_EXPERTS))
